# Initial kernel scaffold; baseline (speedup 1.0000x reference)
import jax
import jax.numpy as jnp
from jax import lax
from jax.experimental import pallas as pl
from jax.experimental.pallas import tpu as pltpu
from jax.experimental.pallas import tpu_sc as plsc

_B, _S, _D = 16384, 77, 64
_NC, _NS = 2, 16
_NW = _NC * _NS
_PW = _B // _NW
_NBUF = 4
_FLUSH = 64
_GPF = _FLUSH // _NBUF
_INV_S = 1.0 / _S
_NCOL = _D // 16


def _body(ids_hbm, tab_hbm, out_hbm, idx_v, rows_v, out_v, *sems):
    wid = lax.axis_index("s") * _NC + lax.axis_index("c")
    base = wid * _PW

    pltpu.sync_copy(ids_hbm.at[pl.ds(base, _PW), :], idx_v)

    def start(e, j):
        pltpu.make_async_copy(
            tab_hbm.at[idx_v.at[e]], rows_v.at[j], sems[j]
        ).start()

    for j in range(_NBUF):
        start(j, j)

    def accumulate(j):
        buf = rows_v.at[j]
        accs = tuple(buf[0, pl.ds(16 * c, 16)] for c in range(_NCOL))

        def rbody(r, accs):
            return tuple(
                accs[c] + buf[r, pl.ds(16 * c, 16)] for c in range(_NCOL)
            )

        return lax.fori_loop(1, _S, rbody, accs, unroll=4)

    def gbody(g, _):
        for j in range(_NBUF):
            e = g * _NBUF + j
            pltpu.make_async_copy(
                tab_hbm.at[idx_v.at[e]], rows_v.at[j], sems[j]
            ).wait()
            accs = accumulate(j)
            row = e % _FLUSH
            for c in range(_NCOL):
                out_v[row, pl.ds(16 * c, 16)] = accs[c] * _INV_S
            nxt = e + _NBUF

            @pl.when(nxt < _PW)
            def _():
                start(nxt, j)

        @pl.when(g % _GPF == _GPF - 1)
        def _():
            row0 = base + (g // _GPF) * _FLUSH
            pltpu.sync_copy(out_v, out_hbm.at[pl.ds(row0, _FLUSH), :])

        return ()

    lax.fori_loop(0, _PW // _NBUF, gbody, ())


@jax.jit
def kernel(text_ids, embedding_table):
    mesh = plsc.VectorSubcoreMesh(
        core_axis_name="c", subcore_axis_name="s",
        num_cores=_NC, num_subcores=_NS,
    )
    f = pl.kernel(
        _body,
        out_type=jax.ShapeDtypeStruct((_B, _D), jnp.float32),
        mesh=mesh,
        scratch_types=[
            pltpu.VMEM((_PW, _S), jnp.int32),
            pltpu.VMEM((_NBUF, _S, _D), jnp.float32),
            pltpu.VMEM((_FLUSH, _D), jnp.float32),
        ] + [pltpu.SemaphoreType.DMA] * _NBUF,
    )
    return f(text_ids, embedding_table)


# speedup vs baseline: 10.0322x; 10.0322x over previous
import jax
import jax.numpy as jnp
from jax import lax
from jax.experimental import pallas as pl
from jax.experimental.pallas import tpu as pltpu
from jax.experimental.pallas import tpu_sc as plsc

_B, _S, _D = 16384, 77, 64
_NC, _NS = 2, 16
_NW = _NC * _NS
_PW = _B // _NW
_NBUF = 4
_FLUSH = 64
_GPF = _FLUSH // _NBUF
_INV_S = 1.0 / _S
_NCOL = _D // 16


def _body(ids_hbm, tab_hbm, out_hbm, idx_v, rows_v, out_v, *sems):
    wid = lax.axis_index("s") * _NC + lax.axis_index("c")
    base = wid * _PW

    pltpu.sync_copy(ids_hbm.at[pl.ds(base, _PW), :], idx_v)

    def start(e, j):
        pltpu.make_async_copy(
            tab_hbm.at[idx_v.at[e]], rows_v.at[j], sems[j]
        ).start()

    for j in range(_NBUF):
        start(j, j)

    def accumulate(j):
        buf = rows_v.at[j]
        accs = tuple(buf[0, pl.ds(16 * c, 16)] for c in range(_NCOL))

        def rbody(r, accs):
            return tuple(
                accs[c] + buf[r, pl.ds(16 * c, 16)] for c in range(_NCOL)
            )

        return lax.fori_loop(1, _S, rbody, accs, unroll=4)

    def gbody(g, _):
        for j in range(_NBUF):
            e = g * _NBUF + j
            pltpu.make_async_copy(
                tab_hbm.at[idx_v.at[e]], rows_v.at[j], sems[j]
            ).wait()
            accs = accumulate(j)
            row = e % _FLUSH
            for c in range(_NCOL):
                out_v[row, pl.ds(16 * c, 16)] = accs[c] * _INV_S
            nxt = e + _NBUF

            @pl.when(nxt < _PW)
            def _():
                start(nxt, j)

        @pl.when(g % _GPF == _GPF - 1)
        def _():
            row0 = base + (g // _GPF) * _FLUSH
            pltpu.sync_copy(out_v, out_hbm.at[pl.ds(row0, _FLUSH), :])

        return ()

    lax.fori_loop(0, _PW // _NBUF, gbody, ())


@jax.jit
def kernel(text_ids, embedding_table):
    mesh = plsc.VectorSubcoreMesh(
        core_axis_name="c", subcore_axis_name="s",
        num_cores=_NC, num_subcores=_NS,
    )
    f = pl.kernel(
        _body,
        out_type=jax.ShapeDtypeStruct((_B, _D), jnp.float32),
        mesh=mesh,
        compiler_params=pltpu.CompilerParams(use_tc_tiling_on_sc=False),
        scratch_types=[
            pltpu.VMEM((_PW, _S), jnp.int32),
            pltpu.VMEM((_NBUF, _S, _D), jnp.float32),
            pltpu.VMEM((_FLUSH, _D), jnp.float32),
        ] + [pltpu.SemaphoreType.DMA] * _NBUF,
    )
    return f(text_ids, embedding_table)


# speedup vs baseline: 21.1236x; 2.1056x over previous
import jax
import jax.numpy as jnp
from jax import lax
from jax.experimental import pallas as pl
from jax.experimental.pallas import tpu as pltpu
from jax.experimental.pallas import tpu_sc as plsc

_B, _S, _D = 16384, 77, 64
_NC, _NS = 2, 16
_NW = _NC * _NS
_PW = _B // _NW
_NBUF = 8
_FLUSH = 64
_GPF = _FLUSH // _NBUF
_INV_S = 1.0 / _S
_NCOL = _D // 16

_V = 1000000
_TBLK = 32768
_TH = _TBLK // 2
_TGRID = (_V + _TBLK - 1) // _TBLK
_VPAD = _TGRID * _TBLK


def _body(ids_hbm, tab_hbm, out_hbm, idx_v, rows_v, out_v, *sems):
    wid = lax.axis_index("s") * _NC + lax.axis_index("c")
    base = wid * _PW

    pltpu.sync_copy(ids_hbm.at[pl.ds(base, _PW), :], idx_v)

    def _g(v):
        return (v & ~(_TBLK - 1)) + 2 * (v & (_TH - 1)) + (
            (v & (_TBLK - 1)) >> (_TBLK.bit_length() - 2))

    def xform(e):
        row = idx_v.at[e]
        for off in (0, 16, 32, 48):
            row[pl.ds(off, 16)] = _g(row[pl.ds(off, 16)])
        v = row[pl.ds(_S - 16, 16)]
        lane = lax.iota(jnp.int32, 16)
        row[pl.ds(_S - 16, 16)] = jnp.where(lane < 3, v, _g(v))

    def start(e, j):
        pltpu.make_async_copy(
            tab_hbm.at[idx_v.at[e]], rows_v.at[j], sems[j]
        ).start()

    for j in range(_NBUF):
        xform(j)
        start(j, j)

    def accumulate(j):
        buf = rows_v.at[j]
        accs = tuple(buf[0, pl.ds(16 * c, 16)] for c in range(_NCOL))

        def rbody(r, accs):
            return tuple(
                accs[c] + buf[r, pl.ds(16 * c, 16)] for c in range(_NCOL)
            )

        return lax.fori_loop(1, _S, rbody, accs, unroll=4)

    def gbody(g, _):
        for j in range(_NBUF):
            e = g * _NBUF + j
            pltpu.make_async_copy(
                tab_hbm.at[idx_v.at[e]], rows_v.at[j], sems[j]
            ).wait()
            accs = accumulate(j)
            row = e % _FLUSH
            for c in range(_NCOL):
                out_v[row, pl.ds(16 * c, 16)] = accs[c] * _INV_S
            nxt = e + _NBUF

            @pl.when(nxt < _PW)
            def _():
                xform(nxt)
                start(nxt, j)

        @pl.when(g % _GPF == _GPF - 1)
        def _():
            row0 = base + (g // _GPF) * _FLUSH
            pltpu.sync_copy(out_v, out_hbm.at[pl.ds(row0, _FLUSH), :])

        return ()

    lax.fori_loop(0, _PW // _NBUF, gbody, ())


def _transpose_body(x_ref, o_ref):
    x = x_ref[...]
    o_ref[:, 0:_D] = x[:, 0:_TH].T
    o_ref[:, _D:2 * _D] = x[:, _TH:_TBLK].T


def _relayout_table(tab_t):
    return pl.pallas_call(
        _transpose_body,
        grid=(_TGRID,),
        in_specs=[pl.BlockSpec((_D, _TBLK), lambda j: (0, j))],
        out_specs=pl.BlockSpec((_TH, 2 * _D), lambda j: (j, 0)),
        out_shape=jax.ShapeDtypeStruct((_TGRID * _TH, 2 * _D), jnp.float32),
    )(tab_t)


@jax.jit
def kernel(text_ids, embedding_table):
    tab_lin = _relayout_table(embedding_table.T).reshape(_VPAD, _D)
    mesh = plsc.VectorSubcoreMesh(
        core_axis_name="c", subcore_axis_name="s",
        num_cores=_NC, num_subcores=_NS,
    )
    f = pl.kernel(
        _body,
        out_type=jax.ShapeDtypeStruct((_B, _D), jnp.float32),
        mesh=mesh,
        compiler_params=pltpu.CompilerParams(use_tc_tiling_on_sc=False),
        scratch_types=[
            pltpu.VMEM((_PW, _S), jnp.int32),
            pltpu.VMEM((_NBUF, _S, _D), jnp.float32),
            pltpu.VMEM((_FLUSH, _D), jnp.float32),
        ] + [pltpu.SemaphoreType.DMA] * _NBUF,
    )
    return f(text_ids, tab_lin)
